# Initial kernel scaffold; baseline (speedup 1.0000x reference)
import jax
import jax.numpy as jnp
from jax.experimental import pallas as pl


def kernel(x, edge_index, batch, post_emb, W1s, W1d, a1s, a1d, b1, W2s, W2d, a2s, a2d, b2, W3s, W3d, a3s, a3d, b3, Wlin, blin):
    raise NotImplementedError("write your pallas kernel here")


# speedup vs baseline: 16.6299x; 16.6299x over previous
import jax
import jax.numpy as jnp
from jax import lax
from jax.experimental import pallas as pl
from jax.experimental.pallas import tpu as pltpu
from jax.experimental.pallas import tpu_sc as plsc

N = 10000
E = 320000
D = 128
H = 128
G = 16
P = 16

NC = 2
NS = 16
NQ = 4
HQ = H // NQ
EPT = E // NS
CHUNK = 80
NCHUNK = EPT // CHUNK
SROWS = 624
BR = 1000
GRID = N // BR

_f32 = jnp.float32


def _dense_tail(h, ws_ref, wd_ref, avs_ref, avd_ref,
                xs_ref, als_ref, ald_ref, mm_ref, i):
    xs = jnp.dot(h, ws_ref[...], preferred_element_type=_f32)
    xd = jnp.dot(h, wd_ref[...], preferred_element_type=_f32)
    for q in range(NQ):
        xs_ref[q] = xs[:, q * HQ:(q + 1) * HQ]
    als = jnp.sum(xs * avs_ref[...], axis=1, keepdims=True)
    ald = jnp.sum(xd * avd_ref[...], axis=1, keepdims=True)
    als_ref[...] = als
    ald_ref[...] = ald
    mloc = jnp.concatenate(
        [jnp.max(als, axis=0, keepdims=True),
         jnp.max(ald, axis=0, keepdims=True)], axis=1)

    @pl.when(i == 0)
    def _():
        mm_ref[...] = mloc

    @pl.when(i > 0)
    def _():
        mm_ref[...] = jnp.maximum(mm_ref[...], mloc)


def _dense_body(h_ref, ws_ref, wd_ref, avs_ref, avd_ref,
                xs_ref, als_ref, ald_ref, mm_ref):
    _dense_tail(h_ref[...], ws_ref, wd_ref, avs_ref, avd_ref,
                xs_ref, als_ref, ald_ref, mm_ref, pl.program_id(0))


_DENSE_OUT_SPECS = [
    pl.BlockSpec((NQ, BR, HQ), lambda i: (0, i, 0)),
    pl.BlockSpec((BR, 1), lambda i: (i, 0)),
    pl.BlockSpec((BR, 1), lambda i: (i, 0)),
    pl.BlockSpec((1, 2), lambda i: (0, 0)),
]
_DENSE_OUT_SHAPE = [
    jax.ShapeDtypeStruct((NQ, N, HQ), _f32),
    jax.ShapeDtypeStruct((N, 1), _f32),
    jax.ShapeDtypeStruct((N, 1), _f32),
    jax.ShapeDtypeStruct((1, 2), _f32),
]
_W_SPEC = pl.BlockSpec((D, H), lambda i: (0, 0))
_V_SPEC = pl.BlockSpec((1, H), lambda i: (0, 0))
_Q_SPEC = pl.BlockSpec((BR, HQ), lambda i: (i, 0))


def _dense_first(x, ws, wd, avs, avd):
    return pl.pallas_call(
        _dense_body,
        grid=(GRID,),
        in_specs=[pl.BlockSpec((BR, D), lambda i: (i, 0)),
                  _W_SPEC, _W_SPEC, _V_SPEC, _V_SPEC],
        out_specs=_DENSE_OUT_SPECS,
        out_shape=_DENSE_OUT_SHAPE,
    )(x, ws, wd, avs.reshape(1, H), avd.reshape(1, H))


def _make_combine_dense(relu):
    def body(p0_ref, p1_ref, p2_ref, p3_ref, dent_ref, b_ref, ws_ref, wd_ref,
             avs_ref, avd_ref, xs_ref, als_ref, ald_ref, mm_ref):
        den = dent_ref[...]
        num = jnp.concatenate(
            [p0_ref[...], p1_ref[...], p2_ref[...], p3_ref[...]], axis=1)
        h = jnp.where(den > 0.0, num / den, 0.0) + b_ref[...]
        if relu:
            h = jnp.maximum(h, 0.0)
        _dense_tail(h, ws_ref, wd_ref, avs_ref, avd_ref,
                    xs_ref, als_ref, ald_ref, mm_ref, pl.program_id(0))

    return body


def _combine_dense(outp, dent, b, ws, wd, avs, avd, relu):
    return pl.pallas_call(
        _make_combine_dense(relu),
        grid=(GRID,),
        in_specs=[
            _Q_SPEC, _Q_SPEC, _Q_SPEC, _Q_SPEC,
            pl.BlockSpec((BR, 1), lambda i: (i, 0)),
            _V_SPEC, _W_SPEC, _W_SPEC, _V_SPEC, _V_SPEC,
        ],
        out_specs=_DENSE_OUT_SPECS,
        out_shape=_DENSE_OUT_SHAPE,
    )(outp[0], outp[1], outp[2], outp[3], dent, b.reshape(1, H), ws, wd,
      avs.reshape(1, H), avd.reshape(1, H))


def _final_body(p0_ref, p1_ref, p2_ref, p3_ref, dent_ref, b_ref, batch_ref,
                pe_ref, wlin_ref, blin_ref, out_ref, pooled_acc, cnt_acc):
    i = pl.program_id(0)
    den = dent_ref[...]
    num = jnp.concatenate(
        [p0_ref[...], p1_ref[...], p2_ref[...], p3_ref[...]], axis=1)
    h = jnp.where(den > 0.0, num / den, 0.0) + b_ref[...]
    bids = batch_ref[...].reshape(1, BR)
    oh = (lax.broadcasted_iota(jnp.int32, (G, BR), 0) == bids).astype(_f32)

    @pl.when(i == 0)
    def _():
        pooled_acc[...] = jnp.zeros((G, H), _f32)
        cnt_acc[...] = jnp.zeros((G, 1), _f32)

    pooled_acc[...] += jnp.dot(oh, h, preferred_element_type=_f32)
    cnt_acc[...] += jnp.sum(oh, axis=1, keepdims=True)

    @pl.when(i == GRID - 1)
    def _():
        pooled = pooled_acc[...] / jnp.maximum(cnt_acc[...], 1.0)
        top = jnp.dot(pooled, wlin_ref[...], preferred_element_type=_f32)
        bot = jnp.dot(pe_ref[...], wlin_ref[...], preferred_element_type=_f32)
        out_ref[...] = jnp.concatenate([top, bot], axis=0) + blin_ref[...]


def _final(outp, dent, b, batch3, post_emb, wlin, blin):
    return pl.pallas_call(
        _final_body,
        grid=(GRID,),
        in_specs=[
            _Q_SPEC, _Q_SPEC, _Q_SPEC, _Q_SPEC,
            pl.BlockSpec((BR, 1), lambda i: (i, 0)),
            _V_SPEC,
            pl.BlockSpec((1, 1, BR), lambda i: (i, 0, 0)),
            pl.BlockSpec((P, H), lambda i: (0, 0)),
            pl.BlockSpec((H, 2), lambda i: (0, 0)),
            pl.BlockSpec((1, 2), lambda i: (0, 0)),
        ],
        out_specs=pl.BlockSpec((G + P, 2), lambda i: (0, 0)),
        out_shape=jax.ShapeDtypeStruct((G + P, 2), _f32),
        scratch_shapes=[
            pltpu.VMEM((G, H), _f32),
            pltpu.VMEM((G, 1), _f32),
        ],
    )(outp[0], outp[1], outp[2], outp[3], dent, b.reshape(1, H), batch3,
      post_emb, wlin, blin.reshape(1, 2))


def _edge_body(src_hbm, dst_hbm, als_hbm, ald_hbm, m_hbm, xs4_hbm,
               outp_hbm, denp_hbm,
               src_v, dst_v, als_v, ald_v, m_v, ex_v, den_v, rowbuf, acc_sh,
               den_sh, gsem, dsem):
    c = lax.axis_index("c")
    s = lax.axis_index("s")

    pltpu.sync_copy(src_hbm.at[s], src_v)
    pltpu.sync_copy(dst_hbm.at[s], dst_v)
    pltpu.sync_copy(als_hbm, als_v)
    pltpu.sync_copy(ald_hbm, ald_v)
    pltpu.sync_copy(m_hbm, m_v)

    z16 = jnp.zeros((16,), _f32)

    @pl.when(s == 0)
    def _():
        def zero_den(i, carry):
            den_v[pl.ds(i * 16, 16)] = z16
            return carry

        lax.fori_loop(0, N // 16, zero_den, 0)
        pltpu.sync_copy(den_v, den_sh)

    def zero_rowbuf(i, carry):
        r = i // (HQ // 16)
        cc = (i % (HQ // 16)) * 16
        rowbuf[r, pl.ds(cc, 16)] = z16
        return carry

    lax.fori_loop(0, CHUNK * (HQ // 16), zero_rowbuf, 0)

    mvec = m_v[...]

    def phase_a(j, carry):
        for k in range(CHUNK // 16):
            off = k * 16
            s16 = src_v[j, pl.ds(off, 16)]
            d16 = dst_v[j, pl.ds(off, 16)]
            a1 = plsc.load_gather(als_v, [s16])
            a2 = plsc.load_gather(ald_v, [d16])
            e = a1 + a2
            e = jnp.where(e >= 0.0, e, e * 0.2)
            ex = jnp.exp(e - mvec)
            ex_v[j, pl.ds(off, 16)] = ex
        return carry

    lax.fori_loop(0, NCHUNK, phase_a, 0)

    plsc.subcore_barrier()

    def den_add(j, carry):
        pltpu.async_copy(ex_v.at[j], den_sh.at[dst_v.at[j]], dsem, add=True)
        return carry

    lax.fori_loop(0, NCHUNK, den_add, 0)

    def drain_den(j, carry):
        pltpu.make_async_copy(ex_v.at[0], den_sh.at[dst_v.at[0]], dsem).wait()
        return carry

    lax.fori_loop(0, NCHUNK, drain_den, 0)

    base = pl.multiple_of(s * SROWS, 8)

    for p in range(2):
        for t in range(7):
            pltpu.sync_copy(rowbuf, acc_sh.at[pl.ds(base + t * CHUNK, CHUNK)])
        pltpu.sync_copy(rowbuf.at[pl.ds(0, SROWS - 7 * CHUNK)],
                        acc_sh.at[pl.ds(base + 7 * CHUNK, SROWS - 7 * CHUNK)])

        @pl.when(s == NS - 1)
        def _():
            pltpu.sync_copy(rowbuf.at[pl.ds(0, N - NS * SROWS)],
                            acc_sh.at[pl.ds(NS * SROWS, N - NS * SROWS)])

        plsc.subcore_barrier()

        q = c * 2 + p

        def phase_b(j, carry):
            pltpu.async_copy(
                xs4_hbm.at[q].at[src_v.at[j]], rowbuf, gsem).wait()

            def scale_group(g, c2):
                exv16 = ex_v[j, pl.ds(g * 16, 16)]
                for l in range(16):
                    r = g * 16 + l
                    sv = jnp.full((16,), exv16[l], _f32)
                    for cb in range(HQ // 16):
                        rowbuf[r, pl.ds(cb * 16, 16)] = (
                            rowbuf[r, pl.ds(cb * 16, 16)] * sv)
                return c2

            lax.fori_loop(0, CHUNK // 16, scale_group, 0)
            pltpu.sync_copy(rowbuf, acc_sh.at[dst_v.at[j]], add=True)
            return carry

        lax.fori_loop(0, NCHUNK, phase_b, 0)

        plsc.subcore_barrier()

        pltpu.sync_copy(acc_sh.at[pl.ds(base, SROWS)],
                        outp_hbm.at[q, pl.ds(base, SROWS)])

        @pl.when(s == NS - 1)
        def _():
            pltpu.sync_copy(acc_sh.at[pl.ds(NS * SROWS, N - NS * SROWS)],
                            outp_hbm.at[q, pl.ds(NS * SROWS, N - NS * SROWS)])

        if p == 0:
            lax.fori_loop(0, CHUNK * (HQ // 16), zero_rowbuf, 0)
            plsc.subcore_barrier()

    @pl.when((c == 0) & (s == 0))
    def _():
        pltpu.sync_copy(den_sh, denp_hbm.at[0, 0])


def _edge(xs4, als, ald, m16, src3, dst3):
    kern = pl.kernel(
        _edge_body,
        out_type=(
            jax.ShapeDtypeStruct((NQ, N, HQ), _f32),
            jax.ShapeDtypeStruct((1, 1, N), _f32),
        ),
        mesh=plsc.VectorSubcoreMesh(
            core_axis_name="c", subcore_axis_name="s",
            num_cores=NC, num_subcores=NS),
        compiler_params=pltpu.CompilerParams(
            needs_layout_passes=False, use_tc_tiling_on_sc=False),
        scratch_types=[
            pltpu.VMEM((NCHUNK, CHUNK), jnp.int32),
            pltpu.VMEM((NCHUNK, CHUNK), jnp.int32),
            pltpu.VMEM((N,), _f32),
            pltpu.VMEM((N,), _f32),
            pltpu.VMEM((16,), _f32),
            pltpu.VMEM((NCHUNK, CHUNK), _f32),
            pltpu.VMEM((N,), _f32),
            pltpu.VMEM((CHUNK, HQ), _f32),
            pltpu.VMEM_SHARED((N, HQ), _f32),
            pltpu.VMEM_SHARED((N,), _f32),
            pltpu.SemaphoreType.DMA,
            pltpu.SemaphoreType.DMA,
        ],
    )
    return kern(src3, dst3, als, ald, m16, xs4)


def kernel(x, edge_index, batch, post_emb,
           W1s, W1d, a1s, a1d, b1,
           W2s, W2d, a2s, a2d, b2,
           W3s, W3d, a3s, a3d, b3,
           Wlin, blin):
    src3 = edge_index[0].astype(jnp.int32).reshape(NS, NCHUNK, CHUNK)
    dst3 = edge_index[1].astype(jnp.int32).reshape(NS, NCHUNK, CHUNK)
    batch3 = batch.astype(jnp.int32).reshape(GRID, 1, BR)

    xs1, als1, ald1, mm1 = _dense_first(x, W1s, W1d, a1s, a1d)
    m16 = jnp.full((16,), mm1[0, 0] + mm1[0, 1], _f32)
    outp1, denp1 = _edge(xs1, als1.reshape(N), ald1.reshape(N), m16, src3, dst3)

    xs2, als2, ald2, mm2 = _combine_dense(
        outp1, denp1.reshape(N, 1), b1, W2s, W2d, a2s, a2d, relu=True)
    m16 = jnp.full((16,), mm2[0, 0] + mm2[0, 1], _f32)
    outp2, denp2 = _edge(xs2, als2.reshape(N), ald2.reshape(N), m16, src3, dst3)

    xs3, als3, ald3, mm3 = _combine_dense(
        outp2, denp2.reshape(N, 1), b2, W3s, W3d, a3s, a3d, relu=True)
    m16 = jnp.full((16,), mm3[0, 0] + mm3[0, 1], _f32)
    outp3, denp3 = _edge(xs3, als3.reshape(N), ald3.reshape(N), m16, src3, dst3)

    return _final(outp3, denp3.reshape(N, 1), b3, batch3, post_emb,
                  Wlin, blin)


# speedup vs baseline: 40.0588x; 2.4088x over previous
import jax
import jax.numpy as jnp
from jax import lax
from jax.experimental import pallas as pl
from jax.experimental.pallas import tpu as pltpu
from jax.experimental.pallas import tpu_sc as plsc

N = 10000
E = 320000
D = 128
H = 128
G = 16
P = 16

NC = 2
NS = 16
NQ = 4
HQ = H // NQ
EPT = E // NS
CHUNK = 80
NCHUNK = EPT // CHUNK
SROWS = 624
BR = 1000
GRID = N // BR

_f32 = jnp.float32


def _dense_tail(h, ws_ref, wd_ref, avs_ref, avd_ref,
                xs_ref, als_ref, ald_ref, mm_ref, i):
    xs = jnp.dot(h, ws_ref[...], preferred_element_type=_f32)
    xd = jnp.dot(h, wd_ref[...], preferred_element_type=_f32)
    for q in range(NQ):
        xs_ref[q] = xs[:, q * HQ:(q + 1) * HQ]
    als = jnp.sum(xs * avs_ref[...], axis=1, keepdims=True)
    ald = jnp.sum(xd * avd_ref[...], axis=1, keepdims=True)
    als_ref[...] = als
    ald_ref[...] = ald
    mloc = jnp.concatenate(
        [jnp.max(als, axis=0, keepdims=True),
         jnp.max(ald, axis=0, keepdims=True)], axis=1)

    @pl.when(i == 0)
    def _():
        mm_ref[...] = mloc

    @pl.when(i > 0)
    def _():
        mm_ref[...] = jnp.maximum(mm_ref[...], mloc)


def _dense_body(h_ref, ws_ref, wd_ref, avs_ref, avd_ref,
                xs_ref, als_ref, ald_ref, mm_ref):
    _dense_tail(h_ref[...], ws_ref, wd_ref, avs_ref, avd_ref,
                xs_ref, als_ref, ald_ref, mm_ref, pl.program_id(0))


_DENSE_OUT_SPECS = [
    pl.BlockSpec((NQ, BR, HQ), lambda i: (0, i, 0)),
    pl.BlockSpec((BR, 1), lambda i: (i, 0)),
    pl.BlockSpec((BR, 1), lambda i: (i, 0)),
    pl.BlockSpec((1, 2), lambda i: (0, 0)),
]
_DENSE_OUT_SHAPE = [
    jax.ShapeDtypeStruct((NQ, N, HQ), _f32),
    jax.ShapeDtypeStruct((N, 1), _f32),
    jax.ShapeDtypeStruct((N, 1), _f32),
    jax.ShapeDtypeStruct((1, 2), _f32),
]
_W_SPEC = pl.BlockSpec((D, H), lambda i: (0, 0))
_V_SPEC = pl.BlockSpec((1, H), lambda i: (0, 0))
_Q_SPEC = pl.BlockSpec((BR, HQ), lambda i: (i, 0))


def _dense_first(x, ws, wd, avs, avd):
    return pl.pallas_call(
        _dense_body,
        grid=(GRID,),
        in_specs=[pl.BlockSpec((BR, D), lambda i: (i, 0)),
                  _W_SPEC, _W_SPEC, _V_SPEC, _V_SPEC],
        out_specs=_DENSE_OUT_SPECS,
        out_shape=_DENSE_OUT_SHAPE,
    )(x, ws, wd, avs.reshape(1, H), avd.reshape(1, H))


def _make_combine_dense(relu):
    def body(p0_ref, p1_ref, p2_ref, p3_ref, dent_ref, b_ref, ws_ref, wd_ref,
             avs_ref, avd_ref, xs_ref, als_ref, ald_ref, mm_ref):
        den = dent_ref[...]
        num = jnp.concatenate(
            [p0_ref[...], p1_ref[...], p2_ref[...], p3_ref[...]], axis=1)
        h = jnp.where(den > 0.0, num / den, 0.0) + b_ref[...]
        if relu:
            h = jnp.maximum(h, 0.0)
        _dense_tail(h, ws_ref, wd_ref, avs_ref, avd_ref,
                    xs_ref, als_ref, ald_ref, mm_ref, pl.program_id(0))

    return body


def _combine_dense(outp, dent, b, ws, wd, avs, avd, relu):
    return pl.pallas_call(
        _make_combine_dense(relu),
        grid=(GRID,),
        in_specs=[
            _Q_SPEC, _Q_SPEC, _Q_SPEC, _Q_SPEC,
            pl.BlockSpec((BR, 1), lambda i: (i, 0)),
            _V_SPEC, _W_SPEC, _W_SPEC, _V_SPEC, _V_SPEC,
        ],
        out_specs=_DENSE_OUT_SPECS,
        out_shape=_DENSE_OUT_SHAPE,
    )(outp[0], outp[1], outp[2], outp[3], dent, b.reshape(1, H), ws, wd,
      avs.reshape(1, H), avd.reshape(1, H))


def _final_body(p0_ref, p1_ref, p2_ref, p3_ref, dent_ref, b_ref, batch_ref,
                pe_ref, wlin_ref, blin_ref, out_ref, pooled_acc, cnt_acc):
    i = pl.program_id(0)
    den = dent_ref[...]
    num = jnp.concatenate(
        [p0_ref[...], p1_ref[...], p2_ref[...], p3_ref[...]], axis=1)
    h = jnp.where(den > 0.0, num / den, 0.0) + b_ref[...]
    bids = batch_ref[...].reshape(1, BR)
    oh = (lax.broadcasted_iota(jnp.int32, (G, BR), 0) == bids).astype(_f32)

    @pl.when(i == 0)
    def _():
        pooled_acc[...] = jnp.zeros((G, H), _f32)
        cnt_acc[...] = jnp.zeros((G, 1), _f32)

    pooled_acc[...] += jnp.dot(oh, h, preferred_element_type=_f32)
    cnt_acc[...] += jnp.sum(oh, axis=1, keepdims=True)

    @pl.when(i == GRID - 1)
    def _():
        pooled = pooled_acc[...] / jnp.maximum(cnt_acc[...], 1.0)
        top = jnp.dot(pooled, wlin_ref[...], preferred_element_type=_f32)
        bot = jnp.dot(pe_ref[...], wlin_ref[...], preferred_element_type=_f32)
        out_ref[...] = jnp.concatenate([top, bot], axis=0) + blin_ref[...]


def _final(outp, dent, b, batch3, post_emb, wlin, blin):
    return pl.pallas_call(
        _final_body,
        grid=(GRID,),
        in_specs=[
            _Q_SPEC, _Q_SPEC, _Q_SPEC, _Q_SPEC,
            pl.BlockSpec((BR, 1), lambda i: (i, 0)),
            _V_SPEC,
            pl.BlockSpec((1, 1, BR), lambda i: (i, 0, 0)),
            pl.BlockSpec((P, H), lambda i: (0, 0)),
            pl.BlockSpec((H, 2), lambda i: (0, 0)),
            pl.BlockSpec((1, 2), lambda i: (0, 0)),
        ],
        out_specs=pl.BlockSpec((G + P, 2), lambda i: (0, 0)),
        out_shape=jax.ShapeDtypeStruct((G + P, 2), _f32),
        scratch_shapes=[
            pltpu.VMEM((G, H), _f32),
            pltpu.VMEM((G, 1), _f32),
        ],
    )(outp[0], outp[1], outp[2], outp[3], dent, b.reshape(1, H), batch3,
      post_emb, wlin, blin.reshape(1, 2))


NB = 5


def _edge_body(src_hbm, dst_hbm, als_hbm, ald_hbm, m_hbm, xs4_hbm,
               outp_hbm, denp_hbm,
               src_v, dst_v, als_v, ald_v, m_v, ex_v, den_v,
               rb0, rb1, rb2, rb3, rb4,
               acc_sh, den_sh,
               gs0, gs1, gs2, gs3, gs4, ss0, ss1, ss2, ss3, ss4, dsem):
    rbs = [rb0, rb1, rb2, rb3, rb4]
    gsems = [gs0, gs1, gs2, gs3, gs4]
    ssems = [ss0, ss1, ss2, ss3, ss4]
    rowbuf = rb0
    c = lax.axis_index("c")
    s = lax.axis_index("s")

    pltpu.sync_copy(src_hbm.at[s], src_v)
    pltpu.sync_copy(dst_hbm.at[s], dst_v)
    pltpu.sync_copy(als_hbm, als_v)
    pltpu.sync_copy(ald_hbm, ald_v)
    pltpu.sync_copy(m_hbm, m_v)

    z16 = jnp.zeros((16,), _f32)

    @pl.when(s == 0)
    def _():
        def zero_den(i, carry):
            den_v[pl.ds(i * 16, 16)] = z16
            return carry

        lax.fori_loop(0, N // 16, zero_den, 0)
        pltpu.sync_copy(den_v, den_sh)

    def zero_rowbuf(i, carry):
        r = i // (HQ // 16)
        cc = (i % (HQ // 16)) * 16
        rowbuf[r, pl.ds(cc, 16)] = z16
        return carry

    lax.fori_loop(0, CHUNK * (HQ // 16), zero_rowbuf, 0)

    mvec = m_v[...]

    def phase_a(j, carry):
        for k in range(CHUNK // 16):
            off = k * 16
            s16 = src_v[j, pl.ds(off, 16)]
            d16 = dst_v[j, pl.ds(off, 16)]
            a1 = plsc.load_gather(als_v, [s16])
            a2 = plsc.load_gather(ald_v, [d16])
            e = a1 + a2
            e = jnp.where(e >= 0.0, e, e * 0.2)
            ex = jnp.exp(e - mvec)
            ex_v[j, pl.ds(off, 16)] = ex
        return carry

    lax.fori_loop(0, NCHUNK, phase_a, 0)

    plsc.subcore_barrier()

    def den_add(j, carry):
        pltpu.async_copy(ex_v.at[j], den_sh.at[dst_v.at[j]], dsem, add=True)
        return carry

    lax.fori_loop(0, NCHUNK, den_add, 0)

    def drain_den(j, carry):
        pltpu.make_async_copy(ex_v.at[0], den_sh.at[dst_v.at[0]], dsem).wait()
        return carry

    lax.fori_loop(0, NCHUNK, drain_den, 0)

    base = pl.multiple_of(s * SROWS, 8)

    def _scale(rb, j):
        for g in range(CHUNK // 16):
            exv16 = ex_v[j, pl.ds(g * 16, 16)]
            for l in range(16):
                r = g * 16 + l
                sv = jnp.full((16,), exv16[l], _f32)
                for cb in range(HQ // 16):
                    rb[r, pl.ds(cb * 16, 16)] = rb[r, pl.ds(cb * 16, 16)] * sv

    for p in range(2):
        if p == 1:
            lax.fori_loop(0, CHUNK * (HQ // 16), zero_rowbuf, 0)

        for t in range(7):
            pltpu.sync_copy(rowbuf, acc_sh.at[pl.ds(base + t * CHUNK, CHUNK)])
        pltpu.sync_copy(rowbuf.at[pl.ds(0, SROWS - 7 * CHUNK)],
                        acc_sh.at[pl.ds(base + 7 * CHUNK, SROWS - 7 * CHUNK)])

        @pl.when(s == NS - 1)
        def _():
            pltpu.sync_copy(rowbuf.at[pl.ds(0, N - NS * SROWS)],
                            acc_sh.at[pl.ds(NS * SROWS, N - NS * SROWS)])

        plsc.subcore_barrier()

        q = c * 2 + p

        for b in range(NB - 1):
            pltpu.async_copy(xs4_hbm.at[q].at[src_v.at[b]], rbs[b], gsems[b])

        def group(jj, carry):
            for b in range(NB):
                j = jj * NB + b
                bp = (b + NB - 1) % NB
                pltpu.make_async_copy(
                    xs4_hbm.at[q].at[src_v.at[j]], rbs[b], gsems[b]).wait()
                _scale(rbs[b], j)
                pltpu.async_copy(
                    rbs[b], acc_sh.at[dst_v.at[j]], ssems[b], add=True)
                if b == 0:
                    @pl.when(jj > 0)
                    def _():
                        pltpu.make_async_copy(
                            rbs[bp], acc_sh.at[dst_v.at[0]], ssems[bp]).wait()
                else:
                    pltpu.make_async_copy(
                        rbs[bp], acc_sh.at[dst_v.at[0]], ssems[bp]).wait()

                @pl.when(j + NB - 1 < NCHUNK)
                def _():
                    pltpu.async_copy(
                        xs4_hbm.at[q].at[src_v.at[j + NB - 1]],
                        rbs[bp], gsems[bp])
            return carry

        lax.fori_loop(0, NCHUNK // NB, group, 0)
        pltpu.make_async_copy(
            rbs[NB - 1], acc_sh.at[dst_v.at[0]], ssems[NB - 1]).wait()

        plsc.subcore_barrier()

        pltpu.sync_copy(acc_sh.at[pl.ds(base, SROWS)],
                        outp_hbm.at[q, pl.ds(base, SROWS)])

        @pl.when(s == NS - 1)
        def _():
            pltpu.sync_copy(acc_sh.at[pl.ds(NS * SROWS, N - NS * SROWS)],
                            outp_hbm.at[q, pl.ds(NS * SROWS, N - NS * SROWS)])

    @pl.when((c == 0) & (s == 0))
    def _():
        pltpu.sync_copy(den_sh, denp_hbm.at[0, 0])


def _edge(xs4, als, ald, m16, src3, dst3):
    kern = pl.kernel(
        _edge_body,
        out_type=(
            jax.ShapeDtypeStruct((NQ, N, HQ), _f32),
            jax.ShapeDtypeStruct((1, 1, N), _f32),
        ),
        mesh=plsc.VectorSubcoreMesh(
            core_axis_name="c", subcore_axis_name="s",
            num_cores=NC, num_subcores=NS),
        compiler_params=pltpu.CompilerParams(
            needs_layout_passes=False, use_tc_tiling_on_sc=False),
        scratch_types=[
            pltpu.VMEM((NCHUNK, CHUNK), jnp.int32),
            pltpu.VMEM((NCHUNK, CHUNK), jnp.int32),
            pltpu.VMEM((N,), _f32),
            pltpu.VMEM((N,), _f32),
            pltpu.VMEM((16,), _f32),
            pltpu.VMEM((NCHUNK, CHUNK), _f32),
            pltpu.VMEM((N,), _f32),
        ] + [pltpu.VMEM((CHUNK, HQ), _f32)] * NB + [
            pltpu.VMEM_SHARED((N, HQ), _f32),
            pltpu.VMEM_SHARED((N,), _f32),
        ] + [pltpu.SemaphoreType.DMA] * (2 * NB + 1),
    )
    return kern(src3, dst3, als, ald, m16, xs4)


def kernel(x, edge_index, batch, post_emb,
           W1s, W1d, a1s, a1d, b1,
           W2s, W2d, a2s, a2d, b2,
           W3s, W3d, a3s, a3d, b3,
           Wlin, blin):
    src3 = edge_index[0].astype(jnp.int32).reshape(NS, NCHUNK, CHUNK)
    dst3 = edge_index[1].astype(jnp.int32).reshape(NS, NCHUNK, CHUNK)
    batch3 = batch.astype(jnp.int32).reshape(GRID, 1, BR)

    xs1, als1, ald1, mm1 = _dense_first(x, W1s, W1d, a1s, a1d)
    m16 = jnp.full((16,), mm1[0, 0] + mm1[0, 1], _f32)
    outp1, denp1 = _edge(xs1, als1.reshape(N), ald1.reshape(N), m16, src3, dst3)

    xs2, als2, ald2, mm2 = _combine_dense(
        outp1, denp1.reshape(N, 1), b1, W2s, W2d, a2s, a2d, relu=True)
    m16 = jnp.full((16,), mm2[0, 0] + mm2[0, 1], _f32)
    outp2, denp2 = _edge(xs2, als2.reshape(N), ald2.reshape(N), m16, src3, dst3)

    xs3, als3, ald3, mm3 = _combine_dense(
        outp2, denp2.reshape(N, 1), b2, W3s, W3d, a3s, a3d, relu=True)
    m16 = jnp.full((16,), mm3[0, 0] + mm3[0, 1], _f32)
    outp3, denp3 = _edge(xs3, als3.reshape(N), ald3.reshape(N), m16, src3, dst3)

    return _final(outp3, denp3.reshape(N, 1), b3, batch3, post_emb,
                  Wlin, blin)


# speedup vs baseline: 40.6193x; 1.0140x over previous
import jax
import jax.numpy as jnp
from jax import lax
from jax.experimental import pallas as pl
from jax.experimental.pallas import tpu as pltpu
from jax.experimental.pallas import tpu_sc as plsc

N = 10000
E = 320000
D = 128
H = 128
G = 16
P = 16

NC = 2
NS = 16
NQ = 4
HQ = H // NQ
EPT = E // NS
CHUNK = 80
NCHUNK = EPT // CHUNK
SROWS = 624
BR = 1000
GRID = N // BR

_f32 = jnp.float32


def _dense_tail(h, ws_ref, wd_ref, avs_ref, avd_ref,
                xs_ref, als_ref, ald_ref, mm_ref, m16_ref, i):
    xs = jnp.dot(h, ws_ref[...], preferred_element_type=_f32)
    xd = jnp.dot(h, wd_ref[...], preferred_element_type=_f32)
    for q in range(NQ):
        xs_ref[q] = xs[:, q * HQ:(q + 1) * HQ]
    als = jnp.sum(xs * avs_ref[...], axis=1, keepdims=True)
    ald = jnp.sum(xd * avd_ref[...], axis=1, keepdims=True)
    als_ref[...] = als
    ald_ref[...] = ald
    mloc = jnp.concatenate(
        [jnp.max(als, axis=0, keepdims=True),
         jnp.max(ald, axis=0, keepdims=True)], axis=1)

    @pl.when(i == 0)
    def _():
        mm_ref[...] = mloc

    @pl.when(i > 0)
    def _():
        mm_ref[...] = jnp.maximum(mm_ref[...], mloc)

    @pl.when(i == GRID - 1)
    def _():
        m16_ref[...] = jnp.full((1, 16), mm_ref[0, 0] + mm_ref[0, 1], _f32)


def _dense_body(h_ref, ws_ref, wd_ref, avs_ref, avd_ref,
                xs_ref, als_ref, ald_ref, mm_ref, m16_ref):
    _dense_tail(h_ref[...], ws_ref, wd_ref, avs_ref, avd_ref,
                xs_ref, als_ref, ald_ref, mm_ref, m16_ref, pl.program_id(0))


_DENSE_OUT_SPECS = [
    pl.BlockSpec((NQ, BR, HQ), lambda i: (0, i, 0)),
    pl.BlockSpec((BR, 1), lambda i: (i, 0)),
    pl.BlockSpec((BR, 1), lambda i: (i, 0)),
    pl.BlockSpec((1, 2), lambda i: (0, 0)),
    pl.BlockSpec((1, 16), lambda i: (0, 0)),
]
_DENSE_OUT_SHAPE = [
    jax.ShapeDtypeStruct((NQ, N, HQ), _f32),
    jax.ShapeDtypeStruct((N, 1), _f32),
    jax.ShapeDtypeStruct((N, 1), _f32),
    jax.ShapeDtypeStruct((1, 2), _f32),
    jax.ShapeDtypeStruct((1, 16), _f32),
]
_W_SPEC = pl.BlockSpec((D, H), lambda i: (0, 0))
_V_SPEC = pl.BlockSpec((1, H), lambda i: (0, 0))
_Q_SPEC = pl.BlockSpec((BR, HQ), lambda i: (i, 0))


def _dense_first(x, ws, wd, avs, avd):
    return pl.pallas_call(
        _dense_body,
        grid=(GRID,),
        in_specs=[pl.BlockSpec((BR, D), lambda i: (i, 0)),
                  _W_SPEC, _W_SPEC, _V_SPEC, _V_SPEC],
        out_specs=_DENSE_OUT_SPECS,
        out_shape=_DENSE_OUT_SHAPE,
    )(x, ws, wd, avs.reshape(1, H), avd.reshape(1, H))


def _make_combine_dense(relu):
    def body(p0_ref, p1_ref, p2_ref, p3_ref, d0_ref, d1_ref, b_ref, ws_ref,
             wd_ref, avs_ref, avd_ref,
             xs_ref, als_ref, ald_ref, mm_ref, m16_ref):
        den = d0_ref[...] + d1_ref[...]
        num = jnp.concatenate(
            [p0_ref[...], p1_ref[...], p2_ref[...], p3_ref[...]], axis=1)
        h = jnp.where(den > 0.0, num / den, 0.0) + b_ref[...]
        if relu:
            h = jnp.maximum(h, 0.0)
        _dense_tail(h, ws_ref, wd_ref, avs_ref, avd_ref,
                    xs_ref, als_ref, ald_ref, mm_ref, m16_ref,
                    pl.program_id(0))

    return body


def _combine_dense(outp, denp, b, ws, wd, avs, avd, relu):
    return pl.pallas_call(
        _make_combine_dense(relu),
        grid=(GRID,),
        in_specs=[
            _Q_SPEC, _Q_SPEC, _Q_SPEC, _Q_SPEC,
            pl.BlockSpec((BR, 1), lambda i: (i, 0)),
            pl.BlockSpec((BR, 1), lambda i: (i, 0)),
            _V_SPEC, _W_SPEC, _W_SPEC, _V_SPEC, _V_SPEC,
        ],
        out_specs=_DENSE_OUT_SPECS,
        out_shape=_DENSE_OUT_SHAPE,
    )(outp[0], outp[1], outp[2], outp[3],
      denp[0].reshape(N, 1), denp[1].reshape(N, 1), b.reshape(1, H), ws, wd,
      avs.reshape(1, H), avd.reshape(1, H))


def _final_body(p0_ref, p1_ref, p2_ref, p3_ref, d0_ref, d1_ref, b_ref,
                batch_ref, pe_ref, wlin_ref, blin_ref, out_ref,
                pooled_acc, cnt_acc):
    i = pl.program_id(0)
    den = d0_ref[...] + d1_ref[...]
    num = jnp.concatenate(
        [p0_ref[...], p1_ref[...], p2_ref[...], p3_ref[...]], axis=1)
    h = jnp.where(den > 0.0, num / den, 0.0) + b_ref[...]
    bids = batch_ref[...].reshape(1, BR)
    oh = (lax.broadcasted_iota(jnp.int32, (G, BR), 0) == bids).astype(_f32)

    @pl.when(i == 0)
    def _():
        pooled_acc[...] = jnp.zeros((G, H), _f32)
        cnt_acc[...] = jnp.zeros((G, 1), _f32)

    pooled_acc[...] += jnp.dot(oh, h, preferred_element_type=_f32)
    cnt_acc[...] += jnp.sum(oh, axis=1, keepdims=True)

    @pl.when(i == GRID - 1)
    def _():
        pooled = pooled_acc[...] / jnp.maximum(cnt_acc[...], 1.0)
        top = jnp.dot(pooled, wlin_ref[...], preferred_element_type=_f32)
        bot = jnp.dot(pe_ref[...], wlin_ref[...], preferred_element_type=_f32)
        out_ref[...] = jnp.concatenate([top, bot], axis=0) + blin_ref[...]


def _final(outp, denp, b, batch3, post_emb, wlin, blin):
    return pl.pallas_call(
        _final_body,
        grid=(GRID,),
        in_specs=[
            _Q_SPEC, _Q_SPEC, _Q_SPEC, _Q_SPEC,
            pl.BlockSpec((BR, 1), lambda i: (i, 0)),
            pl.BlockSpec((BR, 1), lambda i: (i, 0)),
            _V_SPEC,
            pl.BlockSpec((1, 1, BR), lambda i: (i, 0, 0)),
            pl.BlockSpec((P, H), lambda i: (0, 0)),
            pl.BlockSpec((H, 2), lambda i: (0, 0)),
            pl.BlockSpec((1, 2), lambda i: (0, 0)),
        ],
        out_specs=pl.BlockSpec((G + P, 2), lambda i: (0, 0)),
        out_shape=jax.ShapeDtypeStruct((G + P, 2), _f32),
        scratch_shapes=[
            pltpu.VMEM((G, H), _f32),
            pltpu.VMEM((G, 1), _f32),
        ],
    )(outp[0], outp[1], outp[2], outp[3],
      denp[0].reshape(N, 1), denp[1].reshape(N, 1), b.reshape(1, H), batch3,
      post_emb, wlin, blin.reshape(1, 2))


NB = 5


def _edge_body(src_hbm, dst_hbm, als_hbm, ald_hbm, m_hbm, xs4_hbm,
               outp_hbm, denp_hbm,
               src_v, dst_v, als_v, ald_v, m_v, ex_v, den_v,
               rb0, rb1, rb2, rb3, rb4,
               acc_sh, den_sh,
               gs0, gs1, gs2, gs3, gs4, ss0, ss1, ss2, ss3, ss4, dsem):
    rbs = [rb0, rb1, rb2, rb3, rb4]
    gsems = [gs0, gs1, gs2, gs3, gs4]
    ssems = [ss0, ss1, ss2, ss3, ss4]
    rowbuf = rb0
    c = lax.axis_index("c")
    s = lax.axis_index("s")

    pltpu.sync_copy(src_hbm.at[s], src_v)
    pltpu.sync_copy(dst_hbm.at[s], dst_v)
    pltpu.sync_copy(als_hbm, als_v)
    pltpu.sync_copy(ald_hbm, ald_v)
    pltpu.sync_copy(m_hbm, m_v)

    z16 = jnp.zeros((16,), _f32)

    @pl.when(s == 0)
    def _():
        def zero_den(i, carry):
            den_v[pl.ds(i * 16, 16)] = z16
            return carry

        lax.fori_loop(0, N // 16, zero_den, 0)
        pltpu.sync_copy(den_v, den_sh)

    def zero_rowbuf(i, carry):
        r = i // (HQ // 16)
        cc = (i % (HQ // 16)) * 16
        rowbuf[r, pl.ds(cc, 16)] = z16
        return carry

    lax.fori_loop(0, CHUNK * (HQ // 16), zero_rowbuf, 0)

    mvec = m_v[...]

    def phase_a(j, carry):
        for k in range(CHUNK // 16):
            off = k * 16
            s16 = src_v[j, pl.ds(off, 16)]
            d16 = dst_v[j, pl.ds(off, 16)]
            a1 = plsc.load_gather(als_v, [s16])
            a2 = plsc.load_gather(ald_v, [d16])
            e = a1 + a2
            e = jnp.where(e >= 0.0, e, e * 0.2)
            ex = jnp.exp(e - mvec)
            ex_v[j, pl.ds(off, 16)] = ex
        return carry

    lax.fori_loop(0, NCHUNK, phase_a, 0)

    plsc.subcore_barrier()

    dbase = c * (NCHUNK // 2)

    def den_add(i, carry):
        j = dbase + i
        pltpu.async_copy(ex_v.at[j], den_sh.at[dst_v.at[j]], dsem, add=True)
        return carry

    lax.fori_loop(0, NCHUNK // 2, den_add, 0)

    def drain_den(i, carry):
        pltpu.make_async_copy(ex_v.at[0], den_sh.at[dst_v.at[0]], dsem).wait()
        return carry

    lax.fori_loop(0, NCHUNK // 2, drain_den, 0)

    base = pl.multiple_of(s * SROWS, 8)

    lane_consts = [jnp.full((16,), l, jnp.int32) for l in range(16)]

    def _scale(rb, j):
        for g in range(CHUNK // 16):
            exv16 = ex_v[j, pl.ds(g * 16, 16)]
            for l in range(16):
                r = g * 16 + l
                sv = jnp.full((16,), exv16[l], _f32)
                for cb in range(HQ // 16):
                    rb[r, pl.ds(cb * 16, 16)] = rb[r, pl.ds(cb * 16, 16)] * sv

    for p in range(2):
        if p == 1:
            lax.fori_loop(0, CHUNK * (HQ // 16), zero_rowbuf, 0)

        for t in range(7):
            pltpu.sync_copy(rowbuf, acc_sh.at[pl.ds(base + t * CHUNK, CHUNK)])
        pltpu.sync_copy(rowbuf.at[pl.ds(0, SROWS - 7 * CHUNK)],
                        acc_sh.at[pl.ds(base + 7 * CHUNK, SROWS - 7 * CHUNK)])

        @pl.when(s == NS - 1)
        def _():
            pltpu.sync_copy(rowbuf.at[pl.ds(0, N - NS * SROWS)],
                            acc_sh.at[pl.ds(NS * SROWS, N - NS * SROWS)])

        plsc.subcore_barrier()

        q = c * 2 + p

        for b in range(NB - 1):
            pltpu.async_copy(xs4_hbm.at[q].at[src_v.at[b]], rbs[b], gsems[b])

        def group(jj, carry):
            for b in range(NB):
                j = jj * NB + b
                bp = (b + NB - 1) % NB
                pltpu.make_async_copy(
                    xs4_hbm.at[q].at[src_v.at[j]], rbs[b], gsems[b]).wait()
                _scale(rbs[b], j)
                pltpu.async_copy(
                    rbs[b], acc_sh.at[dst_v.at[j]], ssems[b], add=True)
                if b == 0:
                    @pl.when(jj > 0)
                    def _():
                        pltpu.make_async_copy(
                            rbs[bp], acc_sh.at[dst_v.at[0]], ssems[bp]).wait()
                else:
                    pltpu.make_async_copy(
                        rbs[bp], acc_sh.at[dst_v.at[0]], ssems[bp]).wait()

                @pl.when(j + NB - 1 < NCHUNK)
                def _():
                    pltpu.async_copy(
                        xs4_hbm.at[q].at[src_v.at[j + NB - 1]],
                        rbs[bp], gsems[bp])
            return carry

        lax.fori_loop(0, NCHUNK // NB, group, 0)
        pltpu.make_async_copy(
            rbs[NB - 1], acc_sh.at[dst_v.at[0]], ssems[NB - 1]).wait()

        plsc.subcore_barrier()

        pltpu.sync_copy(acc_sh.at[pl.ds(base, SROWS)],
                        outp_hbm.at[q, pl.ds(base, SROWS)])

        @pl.when(s == NS - 1)
        def _():
            pltpu.sync_copy(acc_sh.at[pl.ds(NS * SROWS, N - NS * SROWS)],
                            outp_hbm.at[q, pl.ds(NS * SROWS, N - NS * SROWS)])

    @pl.when(s == 0)
    def _():
        pltpu.sync_copy(den_sh, denp_hbm.at[c, 0])


def _edge(xs4, als, ald, m16, src3, dst3):
    kern = pl.kernel(
        _edge_body,
        out_type=(
            jax.ShapeDtypeStruct((NQ, N, HQ), _f32),
            jax.ShapeDtypeStruct((NC, 1, N), _f32),
        ),
        mesh=plsc.VectorSubcoreMesh(
            core_axis_name="c", subcore_axis_name="s",
            num_cores=NC, num_subcores=NS),
        compiler_params=pltpu.CompilerParams(
            needs_layout_passes=False, use_tc_tiling_on_sc=False),
        scratch_types=[
            pltpu.VMEM((NCHUNK, CHUNK), jnp.int32),
            pltpu.VMEM((NCHUNK, CHUNK), jnp.int32),
            pltpu.VMEM((N,), _f32),
            pltpu.VMEM((N,), _f32),
            pltpu.VMEM((16,), _f32),
            pltpu.VMEM((NCHUNK, CHUNK), _f32),
            pltpu.VMEM((N,), _f32),
        ] + [pltpu.VMEM((CHUNK, HQ), _f32)] * NB + [
            pltpu.VMEM_SHARED((N, HQ), _f32),
            pltpu.VMEM_SHARED((N,), _f32),
        ] + [pltpu.SemaphoreType.DMA] * (2 * NB + 1),
    )
    return kern(src3, dst3, als, ald, m16, xs4)


def kernel(x, edge_index, batch, post_emb,
           W1s, W1d, a1s, a1d, b1,
           W2s, W2d, a2s, a2d, b2,
           W3s, W3d, a3s, a3d, b3,
           Wlin, blin):
    src3 = edge_index[0].astype(jnp.int32).reshape(NS, NCHUNK, CHUNK)
    dst3 = edge_index[1].astype(jnp.int32).reshape(NS, NCHUNK, CHUNK)
    batch3 = batch.astype(jnp.int32).reshape(GRID, 1, BR)

    xs1, als1, ald1, _, m16 = _dense_first(x, W1s, W1d, a1s, a1d)
    outp1, denp1 = _edge(xs1, als1.reshape(N), ald1.reshape(N),
                         m16.reshape(16), src3, dst3)

    xs2, als2, ald2, _, m16 = _combine_dense(
        outp1, denp1, b1, W2s, W2d, a2s, a2d, relu=True)
    outp2, denp2 = _edge(xs2, als2.reshape(N), ald2.reshape(N),
                         m16.reshape(16), src3, dst3)

    xs3, als3, ald3, _, m16 = _combine_dense(
        outp2, denp2, b2, W3s, W3d, a3s, a3d, relu=True)
    outp3, denp3 = _edge(xs3, als3.reshape(N), ald3.reshape(N),
                         m16.reshape(16), src3, dst3)

    return _final(outp3, denp3, b3, batch3, post_emb, Wlin, blin)


# speedup vs baseline: 40.9382x; 1.0079x over previous
import jax
import jax.numpy as jnp
from jax import lax
from jax.experimental import pallas as pl
from jax.experimental.pallas import tpu as pltpu
from jax.experimental.pallas import tpu_sc as plsc

N = 10000
E = 320000
D = 128
H = 128
G = 16
P = 16

NC = 2
NS = 16
NQ = 4
HQ = H // NQ
EPT = E // NS
CHUNK = 80
NCHUNK = EPT // CHUNK
SROWS = 624
BR = 1000
GRID = N // BR

_f32 = jnp.float32


def _dense_tail(h, ws_ref, wd_ref, avs_ref, avd_ref,
                xs_ref, als_ref, ald_ref, mm_ref, m16_ref, i):
    xs = jnp.dot(h, ws_ref[...], preferred_element_type=_f32)
    xd = jnp.dot(h, wd_ref[...], preferred_element_type=_f32)
    for q in range(NQ):
        xs_ref[q] = xs[:, q * HQ:(q + 1) * HQ]
    als = jnp.sum(xs * avs_ref[...], axis=1, keepdims=True)
    ald = jnp.sum(xd * avd_ref[...], axis=1, keepdims=True)
    als_ref[...] = als
    ald_ref[...] = ald
    mloc = jnp.concatenate(
        [jnp.max(als, axis=0, keepdims=True),
         jnp.max(ald, axis=0, keepdims=True)], axis=1)

    @pl.when(i == 0)
    def _():
        mm_ref[...] = mloc

    @pl.when(i > 0)
    def _():
        mm_ref[...] = jnp.maximum(mm_ref[...], mloc)

    @pl.when(i == GRID - 1)
    def _():
        m16_ref[...] = jnp.full((1, 16), mm_ref[0, 0] + mm_ref[0, 1], _f32)


def _dense_body(h_ref, ws_ref, wd_ref, avs_ref, avd_ref,
                xs_ref, als_ref, ald_ref, mm_ref, m16_ref):
    _dense_tail(h_ref[...], ws_ref, wd_ref, avs_ref, avd_ref,
                xs_ref, als_ref, ald_ref, mm_ref, m16_ref, pl.program_id(0))


_DENSE_OUT_SPECS = [
    pl.BlockSpec((NQ, BR, HQ), lambda i: (0, i, 0)),
    pl.BlockSpec((BR, 1), lambda i: (i, 0)),
    pl.BlockSpec((BR, 1), lambda i: (i, 0)),
    pl.BlockSpec((1, 2), lambda i: (0, 0)),
    pl.BlockSpec((1, 16), lambda i: (0, 0)),
]
_DENSE_OUT_SHAPE = [
    jax.ShapeDtypeStruct((NQ, N, HQ), _f32),
    jax.ShapeDtypeStruct((N, 1), _f32),
    jax.ShapeDtypeStruct((N, 1), _f32),
    jax.ShapeDtypeStruct((1, 2), _f32),
    jax.ShapeDtypeStruct((1, 16), _f32),
]
_W_SPEC = pl.BlockSpec((D, H), lambda i: (0, 0))
_V_SPEC = pl.BlockSpec((1, H), lambda i: (0, 0))
_Q_SPEC = pl.BlockSpec((BR, HQ), lambda i: (i, 0))


def _dense_first(x, ws, wd, avs, avd):
    return pl.pallas_call(
        _dense_body,
        grid=(GRID,),
        in_specs=[pl.BlockSpec((BR, D), lambda i: (i, 0)),
                  _W_SPEC, _W_SPEC, _V_SPEC, _V_SPEC],
        out_specs=_DENSE_OUT_SPECS,
        out_shape=_DENSE_OUT_SHAPE,
    )(x, ws, wd, avs.reshape(1, H), avd.reshape(1, H))


def _make_combine_dense(relu):
    def body(p0_ref, p1_ref, p2_ref, p3_ref, d0_ref, d1_ref, b_ref, ws_ref,
             wd_ref, avs_ref, avd_ref,
             xs_ref, als_ref, ald_ref, mm_ref, m16_ref):
        den = d0_ref[...] + d1_ref[...]
        num = jnp.concatenate(
            [p0_ref[...], p1_ref[...], p2_ref[...], p3_ref[...]], axis=1)
        h = jnp.where(den > 0.0, num / den, 0.0) + b_ref[...]
        if relu:
            h = jnp.maximum(h, 0.0)
        _dense_tail(h, ws_ref, wd_ref, avs_ref, avd_ref,
                    xs_ref, als_ref, ald_ref, mm_ref, m16_ref,
                    pl.program_id(0))

    return body


def _combine_dense(outp, denp, b, ws, wd, avs, avd, relu):
    return pl.pallas_call(
        _make_combine_dense(relu),
        grid=(GRID,),
        in_specs=[
            _Q_SPEC, _Q_SPEC, _Q_SPEC, _Q_SPEC,
            pl.BlockSpec((BR, 1), lambda i: (i, 0)),
            pl.BlockSpec((BR, 1), lambda i: (i, 0)),
            _V_SPEC, _W_SPEC, _W_SPEC, _V_SPEC, _V_SPEC,
        ],
        out_specs=_DENSE_OUT_SPECS,
        out_shape=_DENSE_OUT_SHAPE,
    )(outp[0], outp[1], outp[2], outp[3],
      denp[0].reshape(N, 1), denp[1].reshape(N, 1), b.reshape(1, H), ws, wd,
      avs.reshape(1, H), avd.reshape(1, H))


def _final_body(p0_ref, p1_ref, p2_ref, p3_ref, d0_ref, d1_ref, b_ref,
                batch_ref, pe_ref, wlin_ref, blin_ref, out_ref,
                pooled_acc, cnt_acc):
    i = pl.program_id(0)
    den = d0_ref[...] + d1_ref[...]
    num = jnp.concatenate(
        [p0_ref[...], p1_ref[...], p2_ref[...], p3_ref[...]], axis=1)
    h = jnp.where(den > 0.0, num / den, 0.0) + b_ref[...]
    bids = batch_ref[...].reshape(1, BR)
    oh = (lax.broadcasted_iota(jnp.int32, (G, BR), 0) == bids).astype(_f32)

    @pl.when(i == 0)
    def _():
        pooled_acc[...] = jnp.zeros((G, H), _f32)
        cnt_acc[...] = jnp.zeros((G, 1), _f32)

    pooled_acc[...] += jnp.dot(oh, h, preferred_element_type=_f32)
    cnt_acc[...] += jnp.sum(oh, axis=1, keepdims=True)

    @pl.when(i == GRID - 1)
    def _():
        pooled = pooled_acc[...] / jnp.maximum(cnt_acc[...], 1.0)
        top = jnp.dot(pooled, wlin_ref[...], preferred_element_type=_f32)
        bot = jnp.dot(pe_ref[...], wlin_ref[...], preferred_element_type=_f32)
        out_ref[...] = jnp.concatenate([top, bot], axis=0) + blin_ref[...]


def _final(outp, denp, b, batch3, post_emb, wlin, blin):
    return pl.pallas_call(
        _final_body,
        grid=(GRID,),
        in_specs=[
            _Q_SPEC, _Q_SPEC, _Q_SPEC, _Q_SPEC,
            pl.BlockSpec((BR, 1), lambda i: (i, 0)),
            pl.BlockSpec((BR, 1), lambda i: (i, 0)),
            _V_SPEC,
            pl.BlockSpec((1, 1, BR), lambda i: (i, 0, 0)),
            pl.BlockSpec((P, H), lambda i: (0, 0)),
            pl.BlockSpec((H, 2), lambda i: (0, 0)),
            pl.BlockSpec((1, 2), lambda i: (0, 0)),
        ],
        out_specs=pl.BlockSpec((G + P, 2), lambda i: (0, 0)),
        out_shape=jax.ShapeDtypeStruct((G + P, 2), _f32),
        scratch_shapes=[
            pltpu.VMEM((G, H), _f32),
            pltpu.VMEM((G, 1), _f32),
        ],
    )(outp[0], outp[1], outp[2], outp[3],
      denp[0].reshape(N, 1), denp[1].reshape(N, 1), b.reshape(1, H), batch3,
      post_emb, wlin, blin.reshape(1, 2))


NB = 5


def _edge_body(src_hbm, dst_hbm, als_hbm, ald_hbm, m_hbm, xs4_hbm,
               outp_hbm, denp_hbm,
               src_v, dst_v, als_v, ald_v, m_v, ex_v, den_v,
               rb, acc_sh, den_sh, gsem, ssem, dsem):
    rowbuf = rb.at[0]
    c = lax.axis_index("c")
    s = lax.axis_index("s")

    pltpu.sync_copy(src_hbm.at[s], src_v)
    pltpu.sync_copy(dst_hbm.at[s], dst_v)
    pltpu.sync_copy(als_hbm, als_v)
    pltpu.sync_copy(ald_hbm, ald_v)
    pltpu.sync_copy(m_hbm, m_v)

    z16 = jnp.zeros((16,), _f32)

    @pl.when(s == 0)
    def _():
        def zero_den(i, carry):
            den_v[pl.ds(i * 16, 16)] = z16
            return carry

        lax.fori_loop(0, N // 16, zero_den, 0)
        pltpu.sync_copy(den_v, den_sh)

    def zero_rowbuf(i, carry):
        r = i // (HQ // 16)
        cc = (i % (HQ // 16)) * 16
        rb[0, r, pl.ds(cc, 16)] = z16
        return carry

    lax.fori_loop(0, CHUNK * (HQ // 16), zero_rowbuf, 0)

    mvec = m_v[...]

    def phase_a(j, carry):
        for k in range(CHUNK // 16):
            off = k * 16
            s16 = src_v[j, pl.ds(off, 16)]
            d16 = dst_v[j, pl.ds(off, 16)]
            a1 = plsc.load_gather(als_v, [s16])
            a2 = plsc.load_gather(ald_v, [d16])
            e = a1 + a2
            e = jnp.where(e >= 0.0, e, e * 0.2)
            ex = jnp.exp(e - mvec)
            ex_v[j, pl.ds(off, 16)] = ex
        return carry

    lax.fori_loop(0, NCHUNK, phase_a, 0)

    plsc.subcore_barrier()

    dbase = c * (NCHUNK // 2)

    def den_add(i, carry):
        j = dbase + i
        pltpu.async_copy(ex_v.at[j], den_sh.at[dst_v.at[j]], dsem, add=True)
        return carry

    lax.fori_loop(0, NCHUNK // 2, den_add, 0)

    def drain_den(i, carry):
        pltpu.make_async_copy(ex_v.at[0], den_sh.at[dst_v.at[0]], dsem).wait()
        return carry

    lax.fori_loop(0, NCHUNK // 2, drain_den, 0)

    base = pl.multiple_of(s * SROWS, 8)

    for p in range(2):
        if p == 1:
            lax.fori_loop(0, CHUNK * (HQ // 16), zero_rowbuf, 0)

        for t in range(7):
            pltpu.sync_copy(rowbuf, acc_sh.at[pl.ds(base + t * CHUNK, CHUNK)])
        pltpu.sync_copy(rowbuf.at[pl.ds(0, SROWS - 7 * CHUNK)],
                        acc_sh.at[pl.ds(base + 7 * CHUNK, SROWS - 7 * CHUNK)])

        @pl.when(s == NS - 1)
        def _():
            pltpu.sync_copy(rowbuf.at[pl.ds(0, N - NS * SROWS)],
                            acc_sh.at[pl.ds(NS * SROWS, N - NS * SROWS)])

        plsc.subcore_barrier()

        q = c * 2 + p

        for i in range(NB - 1):
            pltpu.async_copy(xs4_hbm.at[q].at[src_v.at[i]], rb.at[i],
                             gsem.at[i])

        def step(j, carry):
            b = lax.rem(j, NB)
            bp = lax.rem(j + NB - 1, NB)
            pltpu.make_async_copy(
                xs4_hbm.at[q].at[src_v.at[j]], rb.at[b], gsem.at[b]).wait()
            for g in range(CHUNK // 16):
                exv16 = ex_v[j, pl.ds(g * 16, 16)]
                for l in range(16):
                    r = g * 16 + l
                    sv = jnp.full((16,), exv16[l], _f32)
                    for cb in range(HQ // 16):
                        rb[b, r, pl.ds(cb * 16, 16)] = (
                            rb[b, r, pl.ds(cb * 16, 16)] * sv)
            pltpu.async_copy(
                rb.at[b], acc_sh.at[dst_v.at[j]], ssem.at[b], add=True)

            @pl.when(j > 0)
            def _():
                pltpu.make_async_copy(
                    rb.at[bp], acc_sh.at[dst_v.at[0]], ssem.at[bp]).wait()

            @pl.when(j + NB - 1 < NCHUNK)
            def _():
                pltpu.async_copy(
                    xs4_hbm.at[q].at[src_v.at[j + NB - 1]],
                    rb.at[bp], gsem.at[bp])
            return carry

        lax.fori_loop(0, NCHUNK, step, 0)
        pltpu.make_async_copy(
            rb.at[(NCHUNK - 1) % NB], acc_sh.at[dst_v.at[0]],
            ssem.at[(NCHUNK - 1) % NB]).wait()

        plsc.subcore_barrier()

        pltpu.sync_copy(acc_sh.at[pl.ds(base, SROWS)],
                        outp_hbm.at[q, pl.ds(base, SROWS)])

        @pl.when(s == NS - 1)
        def _():
            pltpu.sync_copy(acc_sh.at[pl.ds(NS * SROWS, N - NS * SROWS)],
                            outp_hbm.at[q, pl.ds(NS * SROWS, N - NS * SROWS)])

    @pl.when(s == 0)
    def _():
        pltpu.sync_copy(den_sh, denp_hbm.at[c, 0])


def _edge(xs4, als, ald, m16, src3, dst3):
    kern = pl.kernel(
        _edge_body,
        out_type=(
            jax.ShapeDtypeStruct((NQ, N, HQ), _f32),
            jax.ShapeDtypeStruct((NC, 1, N), _f32),
        ),
        mesh=plsc.VectorSubcoreMesh(
            core_axis_name="c", subcore_axis_name="s",
            num_cores=NC, num_subcores=NS),
        compiler_params=pltpu.CompilerParams(
            needs_layout_passes=False, use_tc_tiling_on_sc=False),
        scratch_types=[
            pltpu.VMEM((NCHUNK, CHUNK), jnp.int32),
            pltpu.VMEM((NCHUNK, CHUNK), jnp.int32),
            pltpu.VMEM((N,), _f32),
            pltpu.VMEM((N,), _f32),
            pltpu.VMEM((16,), _f32),
            pltpu.VMEM((NCHUNK, CHUNK), _f32),
            pltpu.VMEM((N,), _f32),
            pltpu.VMEM((NB, CHUNK, HQ), _f32),
            pltpu.VMEM_SHARED((N, HQ), _f32),
            pltpu.VMEM_SHARED((N,), _f32),
            pltpu.SemaphoreType.DMA((NB,)),
            pltpu.SemaphoreType.DMA((NB,)),
            pltpu.SemaphoreType.DMA,
        ],
    )
    return kern(src3, dst3, als, ald, m16, xs4)


def kernel(x, edge_index, batch, post_emb,
           W1s, W1d, a1s, a1d, b1,
           W2s, W2d, a2s, a2d, b2,
           W3s, W3d, a3s, a3d, b3,
           Wlin, blin):
    src3 = edge_index[0].astype(jnp.int32).reshape(NS, NCHUNK, CHUNK)
    dst3 = edge_index[1].astype(jnp.int32).reshape(NS, NCHUNK, CHUNK)
    batch3 = batch.astype(jnp.int32).reshape(GRID, 1, BR)

    xs1, als1, ald1, _, m16 = _dense_first(x, W1s, W1d, a1s, a1d)
    outp1, denp1 = _edge(xs1, als1.reshape(N), ald1.reshape(N),
                         m16.reshape(16), src3, dst3)

    xs2, als2, ald2, _, m16 = _combine_dense(
        outp1, denp1, b1, W2s, W2d, a2s, a2d, relu=True)
    outp2, denp2 = _edge(xs2, als2.reshape(N), ald2.reshape(N),
                         m16.reshape(16), src3, dst3)

    xs3, als3, ald3, _, m16 = _combine_dense(
        outp2, denp2, b2, W3s, W3d, a3s, a3d, relu=True)
    outp3, denp3 = _edge(xs3, als3.reshape(N), ald3.reshape(N),
                         m16.reshape(16), src3, dst3)

    return _final(outp3, denp3, b3, batch3, post_emb, Wlin, blin)
